# baseline (speedup 1.0000x reference)
import dataclasses
import functools

import jax
import jax.numpy as jnp
from jax import lax
from jax.experimental import pallas as pl
from jax.experimental.pallas import tpu as pltpu
from jax.experimental.pallas import tpu_sc as plsc

_N = 10000
_NP = 10000
_E = 160000
_D = 256
_H = 128
_K = 64
_ROWS = 2816
_EPAD = _ROWS * _K
_NC = 2
_NS = 16
_L = 16
_NBUF = 4
_RPT = _ROWS // _NS
_RPW = _ROWS // (_NC * _NS)
_CH = 16
_NPT = _NP // _NS
_NB = 5
_BN = _NP // _NB
_BM = _N // _NB

_mesh = plsc.VectorSubcoreMesh(core_axis_name="c", subcore_axis_name="s")

_sc_params = pltpu.CompilerParams()
if "needs_layout_passes" in pltpu.CompilerParams.__dataclass_fields__:
    _sc_params = dataclasses.replace(_sc_params, needs_layout_passes=False)

_F32 = jnp.float32
_HIGH = jax.lax.Precision.HIGHEST


def _dot(a, b):
    return jax.lax.dot_general(a, b, (((1,), (0,)), ((), ())),
                               preferred_element_type=_F32, precision=_HIGH)


def _mm_body(x_ref, w_ref, o_ref):
    y = _dot(x_ref[...], w_ref[...])
    o_ref[0] = y[:, :_H]
    o_ref[1] = y[:, _H:]


_mm = pl.pallas_call(
    _mm_body,
    grid=(_NB,),
    in_specs=[pl.BlockSpec((_BM, _D), lambda i: (i, 0)),
              pl.BlockSpec((_D, _D), lambda i: (0, 0))],
    out_specs=pl.BlockSpec((2, _BM, _H), lambda i: (0, i, 0)),
    out_shape=jax.ShapeDtypeStruct((2, _N, _H), _F32),
)


@functools.partial(
    pl.kernel, mesh=_mesh,
    out_type=[jax.ShapeDtypeStruct((1, _N), _F32),
              jax.ShapeDtypeStruct((1, _N), _F32)],
    scratch_types=[
        pltpu.VMEM((_RPW, _K), jnp.int32),
        pltpu.VMEM((_RPW, _K), _F32),
        pltpu.VMEM((_N,), _F32),
        pltpu.VMEM_SHARED((_N,), _F32),
    ],
)
def _deg_kernel(d_hbm, w_hbm, out0_hbm, out1_hbm, idx_v, val_v, zero_v,
                acc_sh):
    cid = lax.axis_index("c")
    sid = lax.axis_index("s")
    wid = cid * _NS + sid

    @pl.when(sid == 0)
    def _():
        @pl.loop(0, _N, step=_L)
        def _(i):
            zero_v[pl.ds(i, _L)] = jnp.zeros((_L,), _F32)
        pltpu.sync_copy(zero_v, acc_sh)

    plsc.subcore_barrier()
    pltpu.sync_copy(d_hbm.at[wid], idx_v)
    pltpu.sync_copy(w_hbm.at[wid], val_v)

    @pl.loop(0, _RPW)
    def _(j):
        pltpu.sync_copy(val_v.at[j], acc_sh.at[idx_v.at[j]], add=True)

    plsc.subcore_barrier()

    @pl.when((cid == 0) & (sid == 0))
    def _():
        pltpu.sync_copy(acc_sh, out0_hbm.at[0])

    @pl.when((cid == 1) & (sid == 0))
    def _():
        pltpu.sync_copy(acc_sh, out1_hbm.at[0])


def _dis_body(p0_ref, p1_ref, dis_ref):
    dis_ref[0] = jax.lax.rsqrt(p0_ref[0] + p1_ref[0])


_dis = pl.pallas_call(
    _dis_body, out_shape=jax.ShapeDtypeStruct((1, _N), _F32))


@functools.partial(
    pl.kernel, mesh=_mesh,
    out_type=[jax.ShapeDtypeStruct((_NC, _NS, _NPT, _H), _F32),
              jax.ShapeDtypeStruct((_NP,), _F32)],
    compiler_params=_sc_params,
    scratch_types=[
        pltpu.VMEM((_CH, _K), jnp.int32),
        pltpu.VMEM((_CH, _K), jnp.int32),
        pltpu.VMEM((_CH, _K), _F32),
        pltpu.VMEM((_N,), _F32),
        pltpu.VMEM((_NBUF, _K, _H), _F32),
        pltpu.VMEM((640,), _F32),
        pltpu.VMEM_SHARED((_NP, _H), _F32),
        pltpu.VMEM_SHARED((_NP,), _F32),
        pltpu.SemaphoreType.DMA((_NBUF,)),
        pltpu.SemaphoreType.DMA((_NBUF,)),
    ],
)
def _agg_kernel(s_hbm, d_hbm, w_hbm, dis_hbm, xw_hbm, agg_hbm, c_hbm,
                s_v, d_v, nm_v, dis_v, rb, zero_v, acc_sh, c_sh,
                gsem, ssem):
    cid = lax.axis_index("c")
    sid = lax.axis_index("s")
    pltpu.sync_copy(dis_hbm.at[0], dis_v)

    @pl.loop(0, _K)
    def _(r):
        for g in range(_H // _L):
            rb[0, r, pl.ds(g * _L, _L)] = jnp.zeros((_L,), _F32)

    for t in range(_NPT // _K):
        pltpu.sync_copy(rb.at[0], acc_sh.at[pl.ds(sid * _NPT + t * _K, _K)])
    _TAIL = _NPT - (_NPT // _K) * _K
    if _TAIL:
        pltpu.sync_copy(rb.at[0].at[pl.ds(0, _TAIL)],
                        acc_sh.at[pl.ds(sid * _NPT + _NPT - _TAIL, _TAIL)])

    @pl.when((cid == 0) & (sid == 0))
    def _():
        @pl.loop(0, 640, step=_L)
        def _(i):
            zero_v[pl.ds(i, _L)] = jnp.zeros((_L,), _F32)
        for t in range(_NP // 640):
            pltpu.sync_copy(zero_v, c_sh.at[pl.ds(t * 640, 640)])
        if _NP % 640:
            pltpu.sync_copy(zero_v.at[pl.ds(0, _NP % 640)],
                            c_sh.at[pl.ds((_NP // 640) * 640, _NP % 640)])

    plsc.subcore_barrier()

    def _run(xw_half, do_c):
        @pl.loop(0, _RPT, step=_CH)
        def _(t):
            toff = pl.multiple_of(t, _CH)
            pltpu.sync_copy(s_hbm.at[sid].at[pl.ds(toff, _CH)], s_v)
            pltpu.sync_copy(d_hbm.at[sid].at[pl.ds(toff, _CH)], d_v)
            pltpu.sync_copy(w_hbm.at[sid].at[pl.ds(toff, _CH)], nm_v)

            @pl.loop(0, _CH)
            def _(j):
                for g in range(_K // _L):
                    sl = pl.ds(g * _L, _L)
                    nm_v[j, sl] = (plsc.load_gather(dis_v, [s_v[j, sl]]) *
                                   nm_v[j, sl] *
                                   plsc.load_gather(dis_v, [d_v[j, sl]]))

            if do_c:
                @pl.loop(0, _CH)
                def _(j):
                    pltpu.sync_copy(nm_v.at[j], c_sh.at[s_v.at[j]],
                                    add=True)

            pend_g = {}
            pend_s = {}

            def issue_gather(j):
                b = j % _NBUF
                if j - _NBUF in pend_s:
                    pend_s.pop(j - _NBUF).wait()
                pend_g[j] = pltpu.async_copy(
                    xw_half.at[s_v.at[j]], rb.at[b], gsem.at[b])

            for jp in range(_NBUF - 1):
                issue_gather(jp)
            for j in range(_CH):
                b = j % _NBUF
                if j + _NBUF - 1 < _CH:
                    issue_gather(j + _NBUF - 1)
                pend_g.pop(j).wait()

                @plsc.parallel_loop(0, _K, unroll=8)
                def _(rr):
                    nv = plsc.load_gather(
                        nm_v, [jnp.full((_L,), j, jnp.int32),
                               jnp.full((_L,), rr, jnp.int32)])
                    for g in range(_H // _L):
                        sl = pl.ds(g * _L, _L)
                        rb[b, rr, sl] = rb[b, rr, sl] * nv

                pend_s[j] = pltpu.async_copy(
                    rb.at[b], acc_sh.at[d_v.at[j]], ssem.at[b], add=True)
            for j in list(sorted(pend_s)):
                pend_s.pop(j).wait()

    @pl.when(cid == 0)
    def _():
        _run(xw_hbm.at[0], True)

    @pl.when(cid == 1)
    def _():
        _run(xw_hbm.at[1], False)

    plsc.subcore_barrier()

    pltpu.sync_copy(acc_sh.at[pl.ds(sid * _NPT, _NPT)],
                    agg_hbm.at[cid].at[sid])

    @pl.when((cid == 0) & (sid == 0))
    def _():
        pltpu.sync_copy(c_sh, c_hbm)


def _fin_body(agg_ref, c_ref, b1_ref, w2_ref, b2_ref, o_ref, acc_ref):
    i = pl.program_id(0)

    @pl.when(i == 0)
    def _():
        acc_ref[...] = jnp.zeros_like(acc_ref)

    ct = c_ref[...]
    for h in range(2):
        y = jnp.maximum(agg_ref[h] + b1_ref[h], 0.0)
        acc_ref[pl.ds(h, 1), :] = (acc_ref[pl.ds(h, 1), :] +
                                   jnp.sum(ct * y, axis=0, keepdims=True))

    @pl.when(i == pl.num_programs(0) - 1)
    def _():
        v = acc_ref[...].reshape(1, _D) * (1.0 / _N)
        o_ref[...] = _dot(v, w2_ref[...]) + b2_ref[...]


_fin = pl.pallas_call(
    _fin_body,
    grid=(_NB,),
    in_specs=[
        pl.BlockSpec((2, _BN, _H), lambda i: (0, i, 0)),
        pl.BlockSpec((_BN, 1), lambda i: (i, 0)),
        pl.BlockSpec((2, _H), lambda i: (0, 0)),
        pl.BlockSpec((_D, _D), lambda i: (0, 0)),
        pl.BlockSpec((1, _D), lambda i: (0, 0)),
    ],
    out_specs=pl.BlockSpec((1, _D), lambda i: (0, 0)),
    out_shape=jax.ShapeDtypeStruct((1, _D), _F32),
    scratch_shapes=[pltpu.VMEM((2, _H), _F32)],
)


def kernel(node_features, edge_index, edge_attributes, W1, b1, W2, b2):
    src = edge_index[0].astype(jnp.int32)
    dst = edge_index[1].astype(jnp.int32)
    loop_idx = jnp.arange(_N, dtype=jnp.int32)
    npad = _EPAD - _E - _N
    pad_idx = jnp.arange(npad, dtype=jnp.int32) % _N
    s2 = jnp.concatenate([src, loop_idx, pad_idx])
    d2 = jnp.concatenate([dst, loop_idx, pad_idx])
    e2 = jnp.concatenate([edge_attributes.astype(_F32),
                          jnp.ones((_N,), _F32),
                          jnp.zeros((npad,), _F32)])
    sA = s2.reshape(_NS, _RPT, _K)
    dA = d2.reshape(_NS, _RPT, _K)
    eA = e2.reshape(_NS, _RPT, _K)
    dD = d2.reshape(_NC * _NS, _RPW, _K)
    eD = e2.reshape(_NC * _NS, _RPW, _K)

    xw = _mm(node_features, W1)
    deg0, deg1 = _deg_kernel(dD, eD)
    dis = _dis(deg0, deg1)
    agg, c = _agg_kernel(sA, dA, eA, dis, xw)
    return _fin(agg.reshape(2, _NP, _H), c.reshape(_NP, 1), b1.reshape(2, _H),
                W2, b2.reshape(1, _D))


# speedup vs baseline: 1.0665x; 1.0665x over previous
import dataclasses
import functools

import jax
import jax.numpy as jnp
from jax import lax
from jax.experimental import pallas as pl
from jax.experimental.pallas import tpu as pltpu
from jax.experimental.pallas import tpu_sc as plsc

_N = 10000
_NP = 10000
_E = 160000
_D = 256
_H = 128
_K = 64
_ROWS = 2560
_EPAD = _ROWS * _K
_NC = 2
_NS = 16
_L = 16
_NBUF = 2
_RPT = _ROWS // _NS
_RPW = _ROWS // (_NC * _NS)
_CH = 16
_NPT = _NP // _NS
_NB = 5
_BN = _NP // _NB
_BM = _N // _NB

_mesh = plsc.VectorSubcoreMesh(core_axis_name="c", subcore_axis_name="s")

_sc_params = pltpu.CompilerParams()
if "needs_layout_passes" in pltpu.CompilerParams.__dataclass_fields__:
    _sc_params = dataclasses.replace(_sc_params, needs_layout_passes=False)

_F32 = jnp.float32
_HIGH = jax.lax.Precision.HIGHEST


def _dot(a, b):
    return jax.lax.dot_general(a, b, (((1,), (0,)), ((), ())),
                               preferred_element_type=_F32, precision=_HIGH)


def _mm_body(x_ref, w_ref, o_ref):
    y = _dot(x_ref[...], w_ref[...])
    o_ref[0] = y[:, :_H]
    o_ref[1] = y[:, _H:]


_mm = pl.pallas_call(
    _mm_body,
    grid=(_NB,),
    in_specs=[pl.BlockSpec((_BM, _D), lambda i: (i, 0)),
              pl.BlockSpec((_D, _D), lambda i: (0, 0))],
    out_specs=pl.BlockSpec((2, _BM, _H), lambda i: (0, i, 0)),
    out_shape=jax.ShapeDtypeStruct((2, _N, _H), _F32),
)


@functools.partial(
    pl.kernel, mesh=_mesh,
    out_type=[jax.ShapeDtypeStruct((1, _N), _F32),
              jax.ShapeDtypeStruct((1, _N), _F32)],
    scratch_types=[
        pltpu.VMEM((_RPW, _K), jnp.int32),
        pltpu.VMEM((_RPW, _K), _F32),
        pltpu.VMEM((_N,), _F32),
        pltpu.VMEM_SHARED((_N,), _F32),
    ],
)
def _deg_kernel(d_hbm, w_hbm, out0_hbm, out1_hbm, idx_v, val_v, zero_v,
                acc_sh):
    cid = lax.axis_index("c")
    sid = lax.axis_index("s")
    wid = cid * _NS + sid

    @pl.when(sid == 0)
    def _():
        @pl.loop(0, _N, step=_L)
        def _(i):
            zero_v[pl.ds(i, _L)] = jnp.zeros((_L,), _F32)
        pltpu.sync_copy(zero_v, acc_sh)

    plsc.subcore_barrier()
    pltpu.sync_copy(d_hbm.at[wid], idx_v)
    pltpu.sync_copy(w_hbm.at[wid], val_v)

    @pl.loop(0, _RPW)
    def _(j):
        pltpu.sync_copy(val_v.at[j], acc_sh.at[idx_v.at[j]], add=True)

    plsc.subcore_barrier()

    @pl.when((cid == 0) & (sid == 0))
    def _():
        pltpu.sync_copy(acc_sh, out0_hbm.at[0])

    @pl.when((cid == 1) & (sid == 0))
    def _():
        pltpu.sync_copy(acc_sh, out1_hbm.at[0])


def _dis_body(p0_ref, p1_ref, dis_ref):
    dis_ref[0] = jax.lax.rsqrt(p0_ref[0] + p1_ref[0] + 1.0)


_dis = pl.pallas_call(
    _dis_body, out_shape=jax.ShapeDtypeStruct((1, _N), _F32))


@functools.partial(
    pl.kernel, mesh=_mesh,
    out_type=[jax.ShapeDtypeStruct((_NC, _NS, _NPT, _H), _F32),
              jax.ShapeDtypeStruct((_NP,), _F32)],
    compiler_params=_sc_params,
    scratch_types=[
        pltpu.VMEM((_CH, _K), jnp.int32),
        pltpu.VMEM((_CH, _K), jnp.int32),
        pltpu.VMEM((_CH, _K), _F32),
        pltpu.VMEM((_N,), _F32),
        pltpu.VMEM((_NBUF, _K, _H), _F32),
        pltpu.VMEM((2, _K, _H), _F32),
        pltpu.VMEM((640,), _F32),
        pltpu.VMEM_SHARED((_NP, _H), _F32),
        pltpu.VMEM_SHARED((_NP,), _F32),
        pltpu.SemaphoreType.DMA((_NBUF,)),
        pltpu.SemaphoreType.DMA((2,)),
    ],
)
def _agg_kernel(s_hbm, d_hbm, w_hbm, dis_hbm, xw_hbm, agg_hbm, c_hbm,
                s_v, d_v, nm_v, dis_v, rbh, rbf, zero_v, acc_sh, c_sh,
                gsem, ssem):
    cid = lax.axis_index("c")
    sid = lax.axis_index("s")
    pltpu.sync_copy(dis_hbm.at[0], dis_v)

    @pl.loop(0, _K)
    def _(r):
        for g in range(_H // _L):
            rbf[0, r, pl.ds(g * _L, _L)] = jnp.zeros((_L,), _F32)

    for t in range(_NPT // _K):
        pltpu.sync_copy(rbf.at[0], acc_sh.at[pl.ds(sid * _NPT + t * _K, _K)])
    _TAIL = _NPT - (_NPT // _K) * _K
    if _TAIL:
        pltpu.sync_copy(rbf.at[0].at[pl.ds(0, _TAIL)],
                        acc_sh.at[pl.ds(sid * _NPT + _NPT - _TAIL, _TAIL)])

    @pl.when((cid == 0) & (sid == 0))
    def _():
        @pl.loop(0, 640, step=_L)
        def _(i):
            zero_v[pl.ds(i, _L)] = jnp.zeros((_L,), _F32)
        for t in range(_NP // 640):
            pltpu.sync_copy(zero_v, c_sh.at[pl.ds(t * 640, 640)])
        if _NP % 640:
            pltpu.sync_copy(zero_v.at[pl.ds(0, _NP % 640)],
                            c_sh.at[pl.ds((_NP // 640) * 640, _NP % 640)])

    plsc.subcore_barrier()

    def _run(xw_half, do_c):
        @pl.loop(0, _RPT, step=_CH)
        def _(t):
            toff = pl.multiple_of(t, _CH)
            pltpu.sync_copy(s_hbm.at[sid].at[pl.ds(toff, _CH)], s_v)
            pltpu.sync_copy(d_hbm.at[sid].at[pl.ds(toff, _CH)], d_v)
            pltpu.sync_copy(w_hbm.at[sid].at[pl.ds(toff, _CH)], nm_v)

            @pl.loop(0, _CH)
            def _(j):
                for g in range(_K // _L):
                    sl = pl.ds(g * _L, _L)
                    nm_v[j, sl] = (plsc.load_gather(dis_v, [s_v[j, sl]]) *
                                   nm_v[j, sl] *
                                   plsc.load_gather(dis_v, [d_v[j, sl]]))

            if do_c:
                @pl.loop(0, _CH)
                def _(j):
                    pltpu.sync_copy(nm_v.at[j], c_sh.at[s_v.at[j]],
                                    add=True)

            pend_g = {}
            pend_s = {}

            def issue_gather(j):
                pend_g[j] = pltpu.async_copy(
                    xw_half.at[s_v.at[j]], rbh.at[j % _NBUF],
                    gsem.at[j % _NBUF])

            for jp in range(_NBUF - 1):
                issue_gather(jp)
            for j in range(_CH):
                b = j % _NBUF
                f = j % 2
                if j + _NBUF - 1 < _CH:
                    issue_gather(j + _NBUF - 1)
                pend_g.pop(j).wait()
                if j - 2 in pend_s:
                    pend_s.pop(j - 2).wait()

                @plsc.parallel_loop(0, _K, unroll=8)
                def _(rr):
                    nv = plsc.load_gather(
                        nm_v, [jnp.full((_L,), j, jnp.int32),
                               jnp.full((_L,), rr, jnp.int32)])
                    for g in range(_H // _L):
                        sl = pl.ds(g * _L, _L)
                        rbf[f, rr, sl] = rbh[b, rr, sl] * nv

                pend_s[j] = pltpu.async_copy(
                    rbf.at[f], acc_sh.at[d_v.at[j]], ssem.at[f], add=True)
            for j in list(sorted(pend_s)):
                pend_s.pop(j).wait()

    @pl.when(cid == 0)
    def _():
        _run(xw_hbm.at[0], True)

    @pl.when(cid == 1)
    def _():
        _run(xw_hbm.at[1], False)

    plsc.subcore_barrier()

    pltpu.sync_copy(acc_sh.at[pl.ds(sid * _NPT, _NPT)],
                    agg_hbm.at[cid].at[sid])

    @pl.when((cid == 0) & (sid == 0))
    def _():
        pltpu.sync_copy(c_sh, c_hbm)


def _fin_body(agg_ref, xw_ref, c_ref, dis_ref, b1_ref, w2_ref, b2_ref,
              o_ref, acc_ref):
    i = pl.program_id(0)

    @pl.when(i == 0)
    def _():
        acc_ref[...] = jnp.zeros_like(acc_ref)

    ds2 = dis_ref[...] * dis_ref[...]
    ct = c_ref[...] + ds2
    for h in range(2):
        y = jnp.maximum(agg_ref[h] + ds2 * xw_ref[h] + b1_ref[h], 0.0)
        acc_ref[pl.ds(h, 1), :] = (acc_ref[pl.ds(h, 1), :] +
                                   jnp.sum(ct * y, axis=0, keepdims=True))

    @pl.when(i == pl.num_programs(0) - 1)
    def _():
        v = acc_ref[...].reshape(1, _D) * (1.0 / _N)
        o_ref[...] = _dot(v, w2_ref[...]) + b2_ref[...]


_fin = pl.pallas_call(
    _fin_body,
    grid=(_NB,),
    in_specs=[
        pl.BlockSpec((2, _BN, _H), lambda i: (0, i, 0)),
        pl.BlockSpec((2, _BN, _H), lambda i: (0, i, 0)),
        pl.BlockSpec((_BN, 1), lambda i: (i, 0)),
        pl.BlockSpec((_BN, 1), lambda i: (i, 0)),
        pl.BlockSpec((2, _H), lambda i: (0, 0)),
        pl.BlockSpec((_D, _D), lambda i: (0, 0)),
        pl.BlockSpec((1, _D), lambda i: (0, 0)),
    ],
    out_specs=pl.BlockSpec((1, _D), lambda i: (0, 0)),
    out_shape=jax.ShapeDtypeStruct((1, _D), _F32),
    scratch_shapes=[pltpu.VMEM((2, _H), _F32)],
)


def kernel(node_features, edge_index, edge_attributes, W1, b1, W2, b2):
    src = edge_index[0].astype(jnp.int32)
    dst = edge_index[1].astype(jnp.int32)
    npad = _EPAD - _E
    pad_idx = jnp.arange(npad, dtype=jnp.int32) % _N
    s2 = jnp.concatenate([src, pad_idx])
    d2 = jnp.concatenate([dst, pad_idx])
    e2 = jnp.concatenate([edge_attributes.astype(_F32),
                          jnp.zeros((npad,), _F32)])
    sA = s2.reshape(_NS, _RPT, _K)
    dA = d2.reshape(_NS, _RPT, _K)
    eA = e2.reshape(_NS, _RPT, _K)
    dD = d2.reshape(_NC * _NS, _RPW, _K)
    eD = e2.reshape(_NC * _NS, _RPW, _K)

    xw = _mm(node_features, W1)
    deg0, deg1 = _deg_kernel(dD, eD)
    dis = _dis(deg0, deg1)
    agg, c = _agg_kernel(sA, dA, eA, dis, xw)
    return _fin(agg.reshape(2, _NP, _H), xw, c.reshape(_NP, 1),
                dis.reshape(_NP, 1), b1.reshape(2, _H), W2,
                b2.reshape(1, _D))


# speedup vs baseline: 1.0773x; 1.0101x over previous
import dataclasses
import functools

import jax
import jax.numpy as jnp
from jax import lax
from jax.experimental import pallas as pl
from jax.experimental.pallas import tpu as pltpu
from jax.experimental.pallas import tpu_sc as plsc

_N = 10000
_NP = 10000
_E = 160000
_D = 256
_H = 128
_K = 64
_ROWS = 2560
_EPAD = _ROWS * _K
_NC = 2
_NS = 16
_L = 16
_NBUF = 2
_RPT = _ROWS // _NS
_RPW = _ROWS // (_NC * _NS)
_CH = 16
_NPT = _NP // _NS
_NB = 5
_BN = _NP // _NB
_BM = _N // _NB

_mesh = plsc.VectorSubcoreMesh(core_axis_name="c", subcore_axis_name="s")

_sc_params = pltpu.CompilerParams()
if "needs_layout_passes" in pltpu.CompilerParams.__dataclass_fields__:
    _sc_params = dataclasses.replace(_sc_params, needs_layout_passes=False)

_F32 = jnp.float32
_HIGH = jax.lax.Precision.HIGHEST


def _dot(a, b):
    return jax.lax.dot_general(a, b, (((1,), (0,)), ((), ())),
                               preferred_element_type=_F32, precision=_HIGH)


def _mm_body(x_ref, w_ref, o_ref):
    y = _dot(x_ref[...], w_ref[...])
    o_ref[0] = y[:, :_H]
    o_ref[1] = y[:, _H:]


_mm = pl.pallas_call(
    _mm_body,
    grid=(_NB,),
    in_specs=[pl.BlockSpec((_BM, _D), lambda i: (i, 0)),
              pl.BlockSpec((_D, _D), lambda i: (0, 0))],
    out_specs=pl.BlockSpec((2, _BM, _H), lambda i: (0, i, 0)),
    out_shape=jax.ShapeDtypeStruct((2, _N, _H), _F32),
)


@functools.partial(
    pl.kernel, mesh=_mesh,
    out_type=[jax.ShapeDtypeStruct((1, _N), _F32),
              jax.ShapeDtypeStruct((1, _N), _F32)],
    scratch_types=[
        pltpu.VMEM((_RPW, _K), jnp.int32),
        pltpu.VMEM((_RPW, _K), _F32),
        pltpu.VMEM((_N,), _F32),
        pltpu.VMEM_SHARED((_N,), _F32),
    ],
)
def _deg_kernel(d_hbm, w_hbm, out0_hbm, out1_hbm, idx_v, val_v, zero_v,
                acc_sh):
    cid = lax.axis_index("c")
    sid = lax.axis_index("s")
    wid = cid * _NS + sid

    @pl.when(sid == 0)
    def _():
        @pl.loop(0, _N, step=_L)
        def _(i):
            zero_v[pl.ds(i, _L)] = jnp.zeros((_L,), _F32)
        pltpu.sync_copy(zero_v, acc_sh)

    plsc.subcore_barrier()
    pltpu.sync_copy(d_hbm.at[wid], idx_v)
    pltpu.sync_copy(w_hbm.at[wid], val_v)

    @pl.loop(0, _RPW)
    def _(j):
        pltpu.sync_copy(val_v.at[j], acc_sh.at[idx_v.at[j]], add=True)

    plsc.subcore_barrier()

    @pl.when((cid == 0) & (sid == 0))
    def _():
        pltpu.sync_copy(acc_sh, out0_hbm.at[0])

    @pl.when((cid == 1) & (sid == 0))
    def _():
        pltpu.sync_copy(acc_sh, out1_hbm.at[0])


def _dis_body(p0_ref, p1_ref, dis_ref):
    dis_ref[0] = jax.lax.rsqrt(p0_ref[0] + p1_ref[0] + 1.0)


_dis = pl.pallas_call(
    _dis_body, out_shape=jax.ShapeDtypeStruct((1, _N), _F32))


@functools.partial(
    pl.kernel, mesh=_mesh,
    out_type=[jax.ShapeDtypeStruct((_NC, _NS, _NPT, _H), _F32),
              jax.ShapeDtypeStruct((_NP,), _F32)],
    compiler_params=_sc_params,
    scratch_types=[
        pltpu.VMEM((_CH, _K), jnp.int32),
        pltpu.VMEM((_CH, _K), jnp.int32),
        pltpu.VMEM((_CH, _K), _F32),
        pltpu.VMEM((_N,), _F32),
        pltpu.VMEM((_NBUF, _K, _H), _F32),
        pltpu.VMEM((2, _K, _H), _F32),
        pltpu.VMEM((640,), _F32),
        pltpu.VMEM_SHARED((_NP, _H), _F32),
        pltpu.VMEM_SHARED((_NP,), _F32),
        pltpu.SemaphoreType.DMA((_NBUF, 2)),
        pltpu.SemaphoreType.DMA((2,)),
    ],
)
def _agg_kernel(s_hbm, d_hbm, w_hbm, dis_hbm, xw_hbm, agg_hbm, c_hbm,
                s_v, d_v, nm_v, dis_v, rbh, rbf, zero_v, acc_sh, c_sh,
                gsem, ssem):
    cid = lax.axis_index("c")
    sid = lax.axis_index("s")
    pltpu.sync_copy(dis_hbm.at[0], dis_v)

    @pl.loop(0, _K)
    def _(r):
        for g in range(_H // _L):
            rbf[0, r, pl.ds(g * _L, _L)] = jnp.zeros((_L,), _F32)

    for t in range(_NPT // _K):
        pltpu.sync_copy(rbf.at[0], acc_sh.at[pl.ds(sid * _NPT + t * _K, _K)])
    _TAIL = _NPT - (_NPT // _K) * _K
    if _TAIL:
        pltpu.sync_copy(rbf.at[0].at[pl.ds(0, _TAIL)],
                        acc_sh.at[pl.ds(sid * _NPT + _NPT - _TAIL, _TAIL)])

    @pl.when((cid == 0) & (sid == 0))
    def _():
        @pl.loop(0, 640, step=_L)
        def _(i):
            zero_v[pl.ds(i, _L)] = jnp.zeros((_L,), _F32)
        for t in range(_NP // 640):
            pltpu.sync_copy(zero_v, c_sh.at[pl.ds(t * 640, 640)])
        if _NP % 640:
            pltpu.sync_copy(zero_v.at[pl.ds(0, _NP % 640)],
                            c_sh.at[pl.ds((_NP // 640) * 640, _NP % 640)])

    plsc.subcore_barrier()

    def _run(xw_half, do_c):
        @pl.loop(0, _RPT, step=_CH)
        def _(t):
            toff = pl.multiple_of(t, _CH)
            pltpu.sync_copy(s_hbm.at[sid].at[pl.ds(toff, _CH)], s_v)
            pltpu.sync_copy(d_hbm.at[sid].at[pl.ds(toff, _CH)], d_v)
            pltpu.sync_copy(w_hbm.at[sid].at[pl.ds(toff, _CH)], nm_v)

            @pl.loop(0, _CH)
            def _(j):
                for g in range(_K // _L):
                    sl = pl.ds(g * _L, _L)
                    nm_v[j, sl] = (plsc.load_gather(dis_v, [s_v[j, sl]]) *
                                   nm_v[j, sl] *
                                   plsc.load_gather(dis_v, [d_v[j, sl]]))

            if do_c:
                @pl.loop(0, _CH)
                def _(j):
                    pltpu.sync_copy(nm_v.at[j], c_sh.at[s_v.at[j]],
                                    add=True)

            pend_g = {}
            pend_s = {}

            def issue_gather(j):
                b = j % _NBUF
                hk = _K // 2
                pend_g[j] = [
                    pltpu.async_copy(
                        xw_half.at[s_v.at[j].at[pl.ds(h * hk, hk)]],
                        rbh.at[b].at[pl.ds(h * hk, hk)],
                        gsem.at[b, h])
                    for h in range(2)]

            for jp in range(_NBUF - 1):
                issue_gather(jp)
            for j in range(_CH):
                b = j % _NBUF
                f = j % 2
                if j + _NBUF - 1 < _CH:
                    issue_gather(j + _NBUF - 1)
                for cp in pend_g.pop(j):
                    cp.wait()
                if j - 2 in pend_s:
                    pend_s.pop(j - 2).wait()

                @plsc.parallel_loop(0, _K, unroll=8)
                def _(rr):
                    nv = plsc.load_gather(
                        nm_v, [jnp.full((_L,), j, jnp.int32),
                               jnp.full((_L,), rr, jnp.int32)])
                    for g in range(_H // _L):
                        sl = pl.ds(g * _L, _L)
                        rbf[f, rr, sl] = rbh[b, rr, sl] * nv

                pend_s[j] = pltpu.async_copy(
                    rbf.at[f], acc_sh.at[d_v.at[j]], ssem.at[f], add=True)
            for j in list(sorted(pend_s)):
                pend_s.pop(j).wait()

    @pl.when(cid == 0)
    def _():
        _run(xw_hbm.at[0], True)

    @pl.when(cid == 1)
    def _():
        _run(xw_hbm.at[1], False)

    plsc.subcore_barrier()

    pltpu.sync_copy(acc_sh.at[pl.ds(sid * _NPT, _NPT)],
                    agg_hbm.at[cid].at[sid])

    @pl.when((cid == 0) & (sid == 0))
    def _():
        pltpu.sync_copy(c_sh, c_hbm)


def _fin_body(agg_ref, xw_ref, c_ref, dis_ref, b1_ref, w2_ref, b2_ref,
              o_ref, acc_ref):
    i = pl.program_id(0)

    @pl.when(i == 0)
    def _():
        acc_ref[...] = jnp.zeros_like(acc_ref)

    ds2 = dis_ref[...] * dis_ref[...]
    ct = c_ref[...] + ds2
    for h in range(2):
        y = jnp.maximum(agg_ref[h] + ds2 * xw_ref[h] + b1_ref[h], 0.0)
        acc_ref[pl.ds(h, 1), :] = (acc_ref[pl.ds(h, 1), :] +
                                   jnp.sum(ct * y, axis=0, keepdims=True))

    @pl.when(i == pl.num_programs(0) - 1)
    def _():
        v = acc_ref[...].reshape(1, _D) * (1.0 / _N)
        o_ref[...] = _dot(v, w2_ref[...]) + b2_ref[...]


_fin = pl.pallas_call(
    _fin_body,
    grid=(_NB,),
    in_specs=[
        pl.BlockSpec((2, _BN, _H), lambda i: (0, i, 0)),
        pl.BlockSpec((2, _BN, _H), lambda i: (0, i, 0)),
        pl.BlockSpec((_BN, 1), lambda i: (i, 0)),
        pl.BlockSpec((_BN, 1), lambda i: (i, 0)),
        pl.BlockSpec((2, _H), lambda i: (0, 0)),
        pl.BlockSpec((_D, _D), lambda i: (0, 0)),
        pl.BlockSpec((1, _D), lambda i: (0, 0)),
    ],
    out_specs=pl.BlockSpec((1, _D), lambda i: (0, 0)),
    out_shape=jax.ShapeDtypeStruct((1, _D), _F32),
    scratch_shapes=[pltpu.VMEM((2, _H), _F32)],
)


def kernel(node_features, edge_index, edge_attributes, W1, b1, W2, b2):
    src = edge_index[0].astype(jnp.int32)
    dst = edge_index[1].astype(jnp.int32)
    npad = _EPAD - _E
    pad_idx = jnp.arange(npad, dtype=jnp.int32) % _N
    s2 = jnp.concatenate([src, pad_idx])
    d2 = jnp.concatenate([dst, pad_idx])
    e2 = jnp.concatenate([edge_attributes.astype(_F32),
                          jnp.zeros((npad,), _F32)])
    sA = s2.reshape(_NS, _RPT, _K)
    dA = d2.reshape(_NS, _RPT, _K)
    eA = e2.reshape(_NS, _RPT, _K)
    dD = d2.reshape(_NC * _NS, _RPW, _K)
    eD = e2.reshape(_NC * _NS, _RPW, _K)

    xw = _mm(node_features, W1)
    deg0, deg1 = _deg_kernel(dD, eD)
    dis = _dis(deg0, deg1)
    agg, c = _agg_kernel(sA, dA, eA, dis, xw)
    return _fin(agg.reshape(2, _NP, _H), xw, c.reshape(_NP, 1),
                dis.reshape(_NP, 1), b1.reshape(2, _H), W2,
                b2.reshape(1, _D))


# speedup vs baseline: 1.1224x; 1.0419x over previous
import dataclasses
import functools

import jax
import jax.numpy as jnp
from jax import lax
from jax.experimental import pallas as pl
from jax.experimental.pallas import tpu as pltpu
from jax.experimental.pallas import tpu_sc as plsc

_N = 10000
_NP = 10000
_E = 160000
_D = 256
_H = 128
_K = 64
_ROWS = 2560
_EPAD = _ROWS * _K
_NC = 2
_NS = 16
_L = 16
_NBUF = 2
_RPT = _ROWS // _NS
_RPW = _ROWS // (_NC * _NS)
_CH = 16
_NPT = _NP // _NS
_NB = 5
_BN = _NP // _NB
_BM = _N // _NB

_mesh = plsc.VectorSubcoreMesh(core_axis_name="c", subcore_axis_name="s")

_sc_params = pltpu.CompilerParams()
if "needs_layout_passes" in pltpu.CompilerParams.__dataclass_fields__:
    _sc_params = dataclasses.replace(_sc_params, needs_layout_passes=False)

_F32 = jnp.float32
_HIGH = jax.lax.Precision.HIGHEST


def _dot(a, b):
    return jax.lax.dot_general(a, b, (((1,), (0,)), ((), ())),
                               preferred_element_type=_F32, precision=_HIGH)


def _mm_body(x_ref, w_ref, o_ref):
    y = _dot(x_ref[...], w_ref[...])
    o_ref[0] = y[:, :_H]
    o_ref[1] = y[:, _H:]


_mm = pl.pallas_call(
    _mm_body,
    grid=(_NB,),
    in_specs=[pl.BlockSpec((_BM, _D), lambda i: (i, 0)),
              pl.BlockSpec((_D, _D), lambda i: (0, 0))],
    out_specs=pl.BlockSpec((2, _BM, _H), lambda i: (0, i, 0)),
    out_shape=jax.ShapeDtypeStruct((2, _N, _H), _F32),
)


@functools.partial(
    pl.kernel, mesh=_mesh,
    out_type=[jax.ShapeDtypeStruct((1, _N), _F32),
              jax.ShapeDtypeStruct((1, _N), _F32)],
    scratch_types=[
        pltpu.VMEM((_RPW, _K), jnp.int32),
        pltpu.VMEM((_RPW, _K), _F32),
        pltpu.VMEM((_N,), _F32),
        pltpu.VMEM_SHARED((_N,), _F32),
    ],
)
def _deg_kernel(d_hbm, w_hbm, out0_hbm, out1_hbm, idx_v, val_v, zero_v,
                acc_sh):
    cid = lax.axis_index("c")
    sid = lax.axis_index("s")
    wid = cid * _NS + sid

    @pl.when(sid == 0)
    def _():
        @pl.loop(0, _N, step=_L)
        def _(i):
            zero_v[pl.ds(i, _L)] = jnp.zeros((_L,), _F32)
        pltpu.sync_copy(zero_v, acc_sh)

    plsc.subcore_barrier()
    pltpu.sync_copy(d_hbm.at[wid], idx_v)
    pltpu.sync_copy(w_hbm.at[wid], val_v)

    @pl.loop(0, _RPW)
    def _(j):
        pltpu.sync_copy(val_v.at[j], acc_sh.at[idx_v.at[j]], add=True)

    plsc.subcore_barrier()

    @pl.when((cid == 0) & (sid == 0))
    def _():
        pltpu.sync_copy(acc_sh, out0_hbm.at[0])

    @pl.when((cid == 1) & (sid == 0))
    def _():
        pltpu.sync_copy(acc_sh, out1_hbm.at[0])


def _dis_body(p0_ref, p1_ref, dis_ref):
    dis_ref[0] = jax.lax.rsqrt(p0_ref[0] + p1_ref[0] + 1.0)


_dis = pl.pallas_call(
    _dis_body, out_shape=jax.ShapeDtypeStruct((1, _N), _F32))


@functools.partial(
    pl.kernel, mesh=_mesh,
    out_type=[jax.ShapeDtypeStruct((_NC, _NS, _NPT, _H), _F32),
              jax.ShapeDtypeStruct((_NP,), _F32)],
    compiler_params=_sc_params,
    scratch_types=[
        pltpu.VMEM((_CH, _K), jnp.int32),
        pltpu.VMEM((_CH, _K), jnp.int32),
        pltpu.VMEM((_CH, _K), _F32),
        pltpu.VMEM((_N,), _F32),
        pltpu.VMEM((_NBUF, _K, _H), _F32),
        pltpu.VMEM((2, _K, _H), _F32),
        pltpu.VMEM((640,), _F32),
        pltpu.VMEM_SHARED((_NP, _H), _F32),
        pltpu.VMEM_SHARED((_NP,), _F32),
        pltpu.SemaphoreType.DMA((_NBUF, 2)),
        pltpu.SemaphoreType.DMA((2,)),
        pltpu.SemaphoreType.DMA,
    ],
)
def _agg_kernel(s_hbm, d_hbm, w_hbm, dis_hbm, xw_hbm, agg_hbm, c_hbm,
                s_v, d_v, nm_v, dis_v, rbh, rbf, zero_v, acc_sh, c_sh,
                gsem, ssem, csem):
    cid = lax.axis_index("c")
    sid = lax.axis_index("s")
    pltpu.sync_copy(dis_hbm.at[0], dis_v)

    @pl.loop(0, _K)
    def _(r):
        for g in range(_H // _L):
            rbf[0, r, pl.ds(g * _L, _L)] = jnp.zeros((_L,), _F32)

    for t in range(_NPT // _K):
        pltpu.sync_copy(rbf.at[0], acc_sh.at[pl.ds(sid * _NPT + t * _K, _K)])
    _TAIL = _NPT - (_NPT // _K) * _K
    if _TAIL:
        pltpu.sync_copy(rbf.at[0].at[pl.ds(0, _TAIL)],
                        acc_sh.at[pl.ds(sid * _NPT + _NPT - _TAIL, _TAIL)])

    @pl.when((cid == 0) & (sid == 0))
    def _():
        @pl.loop(0, 640, step=_L)
        def _(i):
            zero_v[pl.ds(i, _L)] = jnp.zeros((_L,), _F32)
        for t in range(_NP // 640):
            pltpu.sync_copy(zero_v, c_sh.at[pl.ds(t * 640, 640)])
        if _NP % 640:
            pltpu.sync_copy(zero_v.at[pl.ds(0, _NP % 640)],
                            c_sh.at[pl.ds((_NP // 640) * 640, _NP % 640)])

    plsc.subcore_barrier()

    def _run(xw_half, do_c):
        @pl.loop(0, _RPT, step=_CH)
        def _(t):
            toff = pl.multiple_of(t, _CH)
            pltpu.sync_copy(s_hbm.at[sid].at[pl.ds(toff, _CH)], s_v)
            pltpu.sync_copy(d_hbm.at[sid].at[pl.ds(toff, _CH)], d_v)
            pltpu.sync_copy(w_hbm.at[sid].at[pl.ds(toff, _CH)], nm_v)

            @pl.loop(0, _CH)
            def _(j):
                for g in range(_K // _L):
                    sl = pl.ds(g * _L, _L)
                    nm_v[j, sl] = (plsc.load_gather(dis_v, [s_v[j, sl]]) *
                                   nm_v[j, sl] *
                                   plsc.load_gather(dis_v, [d_v[j, sl]]))

            pend_c = []
            if do_c:
                for jc in range(_CH):
                    pend_c.append(pltpu.async_copy(
                        nm_v.at[jc], c_sh.at[s_v.at[jc]], csem, add=True))

            pend_g = {}
            pend_s = {}

            def issue_gather(j):
                b = j % _NBUF
                hk = _K // 2
                pend_g[j] = [
                    pltpu.async_copy(
                        xw_half.at[s_v.at[j].at[pl.ds(h * hk, hk)]],
                        rbh.at[b].at[pl.ds(h * hk, hk)],
                        gsem.at[b, h])
                    for h in range(2)]

            for jp in range(_NBUF - 1):
                issue_gather(jp)
            for j in range(_CH):
                b = j % _NBUF
                f = j % 2
                if j + _NBUF - 1 < _CH:
                    issue_gather(j + _NBUF - 1)
                for cp in pend_g.pop(j):
                    cp.wait()
                if j - 2 in pend_s:
                    pend_s.pop(j - 2).wait()

                @plsc.parallel_loop(0, _K, unroll=8)
                def _(rr):
                    nv = plsc.load_gather(
                        nm_v, [jnp.full((_L,), j, jnp.int32),
                               jnp.full((_L,), rr, jnp.int32)])
                    for g in range(_H // _L):
                        sl = pl.ds(g * _L, _L)
                        rbf[f, rr, sl] = rbh[b, rr, sl] * nv

                pend_s[j] = pltpu.async_copy(
                    rbf.at[f], acc_sh.at[d_v.at[j]], ssem.at[f], add=True)
            for j in list(sorted(pend_s)):
                pend_s.pop(j).wait()
            for cp in pend_c:
                cp.wait()

    @pl.when(cid == 0)
    def _():
        _run(xw_hbm.at[0], True)

    @pl.when(cid == 1)
    def _():
        _run(xw_hbm.at[1], False)

    plsc.subcore_barrier()

    pltpu.sync_copy(acc_sh.at[pl.ds(sid * _NPT, _NPT)],
                    agg_hbm.at[cid].at[sid])

    @pl.when((cid == 0) & (sid == 0))
    def _():
        pltpu.sync_copy(c_sh, c_hbm)


def _fin_body(agg_ref, xw_ref, c_ref, dis_ref, b1_ref, w2_ref, b2_ref,
              o_ref, acc_ref):
    i = pl.program_id(0)

    @pl.when(i == 0)
    def _():
        acc_ref[...] = jnp.zeros_like(acc_ref)

    ds2 = dis_ref[...] * dis_ref[...]
    ct = c_ref[...] + ds2
    for h in range(2):
        y = jnp.maximum(agg_ref[h] + ds2 * xw_ref[h] + b1_ref[h], 0.0)
        acc_ref[pl.ds(h, 1), :] = (acc_ref[pl.ds(h, 1), :] +
                                   jnp.sum(ct * y, axis=0, keepdims=True))

    @pl.when(i == pl.num_programs(0) - 1)
    def _():
        v = acc_ref[...].reshape(1, _D) * (1.0 / _N)
        o_ref[...] = _dot(v, w2_ref[...]) + b2_ref[...]


_fin = pl.pallas_call(
    _fin_body,
    grid=(_NB,),
    in_specs=[
        pl.BlockSpec((2, _BN, _H), lambda i: (0, i, 0)),
        pl.BlockSpec((2, _BN, _H), lambda i: (0, i, 0)),
        pl.BlockSpec((_BN, 1), lambda i: (i, 0)),
        pl.BlockSpec((_BN, 1), lambda i: (i, 0)),
        pl.BlockSpec((2, _H), lambda i: (0, 0)),
        pl.BlockSpec((_D, _D), lambda i: (0, 0)),
        pl.BlockSpec((1, _D), lambda i: (0, 0)),
    ],
    out_specs=pl.BlockSpec((1, _D), lambda i: (0, 0)),
    out_shape=jax.ShapeDtypeStruct((1, _D), _F32),
    scratch_shapes=[pltpu.VMEM((2, _H), _F32)],
)


def kernel(node_features, edge_index, edge_attributes, W1, b1, W2, b2):
    src = edge_index[0].astype(jnp.int32)
    dst = edge_index[1].astype(jnp.int32)
    npad = _EPAD - _E
    pad_idx = jnp.arange(npad, dtype=jnp.int32) % _N
    s2 = jnp.concatenate([src, pad_idx])
    d2 = jnp.concatenate([dst, pad_idx])
    e2 = jnp.concatenate([edge_attributes.astype(_F32),
                          jnp.zeros((npad,), _F32)])
    sA = s2.reshape(_NS, _RPT, _K)
    dA = d2.reshape(_NS, _RPT, _K)
    eA = e2.reshape(_NS, _RPT, _K)
    dD = d2.reshape(_NC * _NS, _RPW, _K)
    eD = e2.reshape(_NC * _NS, _RPW, _K)

    xw = _mm(node_features, W1)
    deg0, deg1 = _deg_kernel(dD, eD)
    dis = _dis(deg0, deg1)
    agg, c = _agg_kernel(sA, dA, eA, dis, xw)
    return _fin(agg.reshape(2, _NP, _H), xw, c.reshape(_NP, 1),
                dis.reshape(_NP, 1), b1.reshape(2, _H), W2,
                b2.reshape(1, _D))


# speedup vs baseline: 1.1404x; 1.0160x over previous
import dataclasses
import functools

import jax
import jax.numpy as jnp
from jax import lax
from jax.experimental import pallas as pl
from jax.experimental.pallas import tpu as pltpu
from jax.experimental.pallas import tpu_sc as plsc

_N = 10000
_NP = 10000
_E = 160000
_D = 256
_H = 128
_K = 64
_ROWS = 2560
_EPAD = _ROWS * _K
_NC = 2
_NS = 16
_L = 16
_NBUF = 2
_RPT = _ROWS // _NS
_RPW = _ROWS // (_NC * _NS)
_CH = 16
_NPT = _NP // _NS
_NB = 5
_BN = _NP // _NB
_BM = _N // _NB

_mesh = plsc.VectorSubcoreMesh(core_axis_name="c", subcore_axis_name="s")

_sc_params = pltpu.CompilerParams()
if "needs_layout_passes" in pltpu.CompilerParams.__dataclass_fields__:
    _sc_params = dataclasses.replace(_sc_params, needs_layout_passes=False)

_F32 = jnp.float32
_HIGH = jax.lax.Precision.HIGHEST


def _dot(a, b):
    return jax.lax.dot_general(a, b, (((1,), (0,)), ((), ())),
                               preferred_element_type=_F32, precision=_HIGH)


def _mm_body(x_ref, w_ref, o_ref):
    y = _dot(x_ref[...], w_ref[...])
    o_ref[0] = y[:, :_H]
    o_ref[1] = y[:, _H:]


_mm = pl.pallas_call(
    _mm_body,
    grid=(_NB,),
    in_specs=[pl.BlockSpec((_BM, _D), lambda i: (i, 0)),
              pl.BlockSpec((_D, _D), lambda i: (0, 0))],
    out_specs=pl.BlockSpec((2, _BM, _H), lambda i: (0, i, 0)),
    out_shape=jax.ShapeDtypeStruct((2, _N, _H), _F32),
)


@functools.partial(
    pl.kernel, mesh=_mesh,
    out_type=[jax.ShapeDtypeStruct((1, _N), _F32),
              jax.ShapeDtypeStruct((1, _N), _F32)],
    scratch_types=[
        pltpu.VMEM((_RPW, _K), jnp.int32),
        pltpu.VMEM((_RPW, _K), _F32),
        pltpu.VMEM((_N,), _F32),
        pltpu.VMEM_SHARED((_N,), _F32),
    ],
)
def _deg_kernel(d_hbm, w_hbm, out0_hbm, out1_hbm, idx_v, val_v, zero_v,
                acc_sh):
    cid = lax.axis_index("c")
    sid = lax.axis_index("s")
    wid = cid * _NS + sid

    @pl.when(sid == 0)
    def _():
        @pl.loop(0, _N, step=_L)
        def _(i):
            zero_v[pl.ds(i, _L)] = jnp.zeros((_L,), _F32)
        pltpu.sync_copy(zero_v, acc_sh)

    plsc.subcore_barrier()
    pltpu.sync_copy(d_hbm.at[wid], idx_v)
    pltpu.sync_copy(w_hbm.at[wid], val_v)

    @pl.loop(0, _RPW)
    def _(j):
        pltpu.sync_copy(val_v.at[j], acc_sh.at[idx_v.at[j]], add=True)

    plsc.subcore_barrier()

    @pl.when((cid == 0) & (sid == 0))
    def _():
        pltpu.sync_copy(acc_sh, out0_hbm.at[0])

    @pl.when((cid == 1) & (sid == 0))
    def _():
        pltpu.sync_copy(acc_sh, out1_hbm.at[0])


def _dis_body(p0_ref, p1_ref, dis_ref):
    dis_ref[0] = jax.lax.rsqrt(p0_ref[0] + p1_ref[0] + 1.0)


_dis = pl.pallas_call(
    _dis_body, out_shape=jax.ShapeDtypeStruct((1, _N), _F32))


@functools.partial(
    pl.kernel, mesh=_mesh,
    out_type=[jax.ShapeDtypeStruct((_NC, _NS, _NPT, _H), _F32),
              jax.ShapeDtypeStruct((_NP,), _F32)],
    compiler_params=_sc_params,
    scratch_types=[
        pltpu.VMEM((_CH, _K), jnp.int32),
        pltpu.VMEM((_CH, _K), jnp.int32),
        pltpu.VMEM((_CH, _K), _F32),
        pltpu.VMEM((_N,), _F32),
        pltpu.VMEM((_NBUF, _K, _H), _F32),
        pltpu.VMEM((2, _K, _H), _F32),
        pltpu.VMEM((640,), _F32),
        pltpu.VMEM_SHARED((_NP, _H), _F32),
        pltpu.VMEM_SHARED((_NP,), _F32),
        pltpu.SemaphoreType.DMA((_NBUF, 2)),
        pltpu.SemaphoreType.DMA((2,)),
        pltpu.SemaphoreType.DMA,
    ],
)
def _agg_kernel(s_hbm, d_hbm, w_hbm, dis_hbm, xw_hbm, agg_hbm, c_hbm,
                s_v, d_v, nm_v, dis_v, rbh, rbf, zero_v, acc_sh, c_sh,
                gsem, ssem, csem):
    cid = lax.axis_index("c")
    sid = lax.axis_index("s")
    pltpu.sync_copy(dis_hbm.at[0], dis_v)

    @pl.loop(0, _K)
    def _(r):
        for g in range(_H // _L):
            rbf[0, r, pl.ds(g * _L, _L)] = jnp.zeros((_L,), _F32)

    for t in range(_NPT // _K):
        pltpu.sync_copy(rbf.at[0], acc_sh.at[pl.ds(sid * _NPT + t * _K, _K)])
    _TAIL = _NPT - (_NPT // _K) * _K
    if _TAIL:
        pltpu.sync_copy(rbf.at[0].at[pl.ds(0, _TAIL)],
                        acc_sh.at[pl.ds(sid * _NPT + _NPT - _TAIL, _TAIL)])

    @pl.when((cid == 0) & (sid == 0))
    def _():
        @pl.loop(0, 640, step=_L)
        def _(i):
            zero_v[pl.ds(i, _L)] = jnp.zeros((_L,), _F32)
        for t in range(_NP // 640):
            pltpu.sync_copy(zero_v, c_sh.at[pl.ds(t * 640, 640)])
        if _NP % 640:
            pltpu.sync_copy(zero_v.at[pl.ds(0, _NP % 640)],
                            c_sh.at[pl.ds((_NP // 640) * 640, _NP % 640)])

    plsc.subcore_barrier()

    def _run(xw_half, do_c):
        @pl.loop(0, _RPT, step=_CH)
        def _(t):
            toff = pl.multiple_of(t, _CH)
            pltpu.sync_copy(s_hbm.at[sid].at[pl.ds(toff, _CH)], s_v)
            pltpu.sync_copy(d_hbm.at[sid].at[pl.ds(toff, _CH)], d_v)
            pltpu.sync_copy(w_hbm.at[sid].at[pl.ds(toff, _CH)], nm_v)

            @pl.loop(0, _CH)
            def _(j):
                for g in range(_K // _L):
                    sl = pl.ds(g * _L, _L)
                    nm_v[j, sl] = (plsc.load_gather(dis_v, [s_v[j, sl]]) *
                                   nm_v[j, sl] *
                                   plsc.load_gather(dis_v, [d_v[j, sl]]))

            pend_c = []
            if do_c:
                for jc in range(_CH):
                    pend_c.append(pltpu.async_copy(
                        nm_v.at[jc], c_sh.at[s_v.at[jc]], csem, add=True))

            pend_g = {}
            pend_s = {}

            def issue_gather(j):
                b = j % _NBUF
                hk = _K // 2
                pend_g[j] = [
                    pltpu.async_copy(
                        xw_half.at[s_v.at[j].at[pl.ds(h * hk, hk)]],
                        rbh.at[b].at[pl.ds(h * hk, hk)],
                        gsem.at[b, h])
                    for h in range(2)]

            for jp in range(_NBUF - 1):
                issue_gather(jp)
            for j in range(_CH):
                b = j % _NBUF
                f = j % 2
                if j + _NBUF - 1 < _CH:
                    issue_gather(j + _NBUF - 1)
                for cp in pend_g.pop(j):
                    cp.wait()
                if j - 2 in pend_s:
                    pend_s.pop(j - 2).wait()

                @plsc.parallel_loop(0, _K, unroll=8)
                def _(rr):
                    grp = nm_v[j, pl.ds((rr // _L) * _L, _L)]
                    nv = grp.at[jnp.full((_L,), rr % _L, jnp.int32)].get(
                        mode="promise_in_bounds")
                    for g in range(_H // _L):
                        sl = pl.ds(g * _L, _L)
                        rbf[f, rr, sl] = rbh[b, rr, sl] * nv

                pend_s[j] = pltpu.async_copy(
                    rbf.at[f], acc_sh.at[d_v.at[j]], ssem.at[f], add=True)
            for j in list(sorted(pend_s)):
                pend_s.pop(j).wait()
            for cp in pend_c:
                cp.wait()

    @pl.when(cid == 0)
    def _():
        _run(xw_hbm.at[0], True)

    @pl.when(cid == 1)
    def _():
        _run(xw_hbm.at[1], False)

    plsc.subcore_barrier()

    pltpu.sync_copy(acc_sh.at[pl.ds(sid * _NPT, _NPT)],
                    agg_hbm.at[cid].at[sid])

    @pl.when((cid == 0) & (sid == 0))
    def _():
        pltpu.sync_copy(c_sh, c_hbm)


def _fin_body(agg_ref, xw_ref, c_ref, dis_ref, b1_ref, w2_ref, b2_ref,
              o_ref, acc_ref):
    i = pl.program_id(0)

    @pl.when(i == 0)
    def _():
        acc_ref[...] = jnp.zeros_like(acc_ref)

    ds2 = dis_ref[...] * dis_ref[...]
    ct = c_ref[...] + ds2
    for h in range(2):
        y = jnp.maximum(agg_ref[h] + ds2 * xw_ref[h] + b1_ref[h], 0.0)
        acc_ref[pl.ds(h, 1), :] = (acc_ref[pl.ds(h, 1), :] +
                                   jnp.sum(ct * y, axis=0, keepdims=True))

    @pl.when(i == pl.num_programs(0) - 1)
    def _():
        v = acc_ref[...].reshape(1, _D) * (1.0 / _N)
        o_ref[...] = _dot(v, w2_ref[...]) + b2_ref[...]


_fin = pl.pallas_call(
    _fin_body,
    grid=(_NB,),
    in_specs=[
        pl.BlockSpec((2, _BN, _H), lambda i: (0, i, 0)),
        pl.BlockSpec((2, _BN, _H), lambda i: (0, i, 0)),
        pl.BlockSpec((_BN, 1), lambda i: (i, 0)),
        pl.BlockSpec((_BN, 1), lambda i: (i, 0)),
        pl.BlockSpec((2, _H), lambda i: (0, 0)),
        pl.BlockSpec((_D, _D), lambda i: (0, 0)),
        pl.BlockSpec((1, _D), lambda i: (0, 0)),
    ],
    out_specs=pl.BlockSpec((1, _D), lambda i: (0, 0)),
    out_shape=jax.ShapeDtypeStruct((1, _D), _F32),
    scratch_shapes=[pltpu.VMEM((2, _H), _F32)],
)


def kernel(node_features, edge_index, edge_attributes, W1, b1, W2, b2):
    src = edge_index[0].astype(jnp.int32)
    dst = edge_index[1].astype(jnp.int32)
    npad = _EPAD - _E
    pad_idx = jnp.arange(npad, dtype=jnp.int32) % _N
    s2 = jnp.concatenate([src, pad_idx])
    d2 = jnp.concatenate([dst, pad_idx])
    e2 = jnp.concatenate([edge_attributes.astype(_F32),
                          jnp.zeros((npad,), _F32)])
    sA = s2.reshape(_NS, _RPT, _K)
    dA = d2.reshape(_NS, _RPT, _K)
    eA = e2.reshape(_NS, _RPT, _K)
    dD = d2.reshape(_NC * _NS, _RPW, _K)
    eD = e2.reshape(_NC * _NS, _RPW, _K)

    xw = _mm(node_features, W1)
    deg0, deg1 = _deg_kernel(dD, eD)
    dis = _dis(deg0, deg1)
    agg, c = _agg_kernel(sA, dA, eA, dis, xw)
    return _fin(agg.reshape(2, _NP, _H), xw, c.reshape(_NP, 1),
                dis.reshape(_NP, 1), b1.reshape(2, _H), W2,
                b2.reshape(1, _D))
